# baseline (speedup 1.0000x reference)
import functools
import jax
import jax.numpy as jnp
from jax import lax
from jax.experimental import pallas as pl
from jax.experimental.pallas import tpu as pltpu
from jax.experimental.pallas import tpu_sc as plsc

B, N, F = 8, 128, 128
C1, C2 = 256, 128
NSLICE = 32


def _tanh(x):
    x = jnp.minimum(jnp.maximum(x, -7.90531110763549805), 7.90531110763549805)
    x2 = x * x
    p = 2.00018790482477e-13 + x2 * -2.76076847742355e-16
    p = -8.60467152213735e-11 + x2 * p
    p = 5.12229709037114e-08 + x2 * p
    p = 1.48572235717979e-05 + x2 * p
    p = 6.37261928875436e-04 + x2 * p
    p = 4.89352455891786e-03 + x2 * p
    p = x * p
    q = 1.19825839466702e-06
    q = 1.18534705686654e-04 + x2 * q
    q = 2.26843463243900e-03 + x2 * q
    q = 4.89352518554385e-03 + x2 * q
    return p / q


def _splat(v, lane):
    idx = jnp.full((16,), lane, jnp.int32)
    return v.at[idx].get(mode="promise_in_bounds")


def _hsum(v):
    lane = lax.iota(jnp.int32, 16)
    for k in (1, 2, 4, 8):
        v = v + v.at[lane ^ k].get(mode="promise_in_bounds")
    return v


def _hmax(v):
    lane = lax.iota(jnp.int32, 16)
    for k in (1, 2, 4, 8):
        v = jnp.maximum(v, v.at[lane ^ k].get(mode="promise_in_bounds"))
    return v


def _sc_call(xT, adjT, k1t, k2t, Wd1, bd1, Wd2p, bd2p):
    mesh = plsc.VectorSubcoreMesh(core_axis_name="c", subcore_axis_name="s",
                                  num_cores=2, num_subcores=16)

    @functools.partial(
        pl.kernel,
        mesh=mesh,
        out_type=jax.ShapeDtypeStruct((B, 16), jnp.float32),
        scratch_types=[
            pltpu.VMEM((F, N), jnp.float32),
            pltpu.VMEM((N, N), jnp.float32),
            pltpu.VMEM((3, C1), jnp.float32),
            pltpu.VMEM((2, C2), jnp.float32),
            pltpu.VMEM((128, 64), jnp.float32),
            pltpu.VMEM((64,), jnp.float32),
            pltpu.VMEM((64, 16), jnp.float32),
            pltpu.VMEM((16,), jnp.float32),
            pltpu.VMEM((NSLICE,), jnp.float32),
            pltpu.VMEM((N,), jnp.float32),
            pltpu.VMEM((NSLICE,), jnp.float32),
            pltpu.VMEM((N,), jnp.float32),
            pltpu.VMEM((NSLICE,), jnp.float32),
            pltpu.VMEM((C2,), jnp.float32),
            pltpu.VMEM((C2,), jnp.float32),
            pltpu.VMEM((C2,), jnp.float32),
            pltpu.VMEM((64,), jnp.float32),
            pltpu.VMEM((16,), jnp.float32),
            pltpu.VMEM_SHARED((4, N), jnp.float32),
            pltpu.VMEM_SHARED((4, N), jnp.float32),
            pltpu.VMEM_SHARED((16, C2), jnp.float32),
        ],
    )
    def k(xT_hbm, adjT_hbm, k1_hbm, k2_hbm, wd1_hbm, bd1_hbm, wd2_hbm,
          bd2_hbm, out_hbm, xloc, aloc, k1loc, k2loc, wd1loc, bd1loc,
          wd2loc, bd2loc, xsbuf, xs_b, h1buf, h1s_b, wbuf, pbuf, psum,
          ptmp, d1buf, obuf, xs_sh, h1s_sh, pool_sh):
        cid = lax.axis_index("c")
        sid = lax.axis_index("s")
        b = cid * 4 + sid // 4
        q = sid % 4
        lb = sid // 4
        nbase = q * NSLICE
        z16 = jnp.zeros((16,), jnp.float32)

        pltpu.sync_copy(xT_hbm.at[b], xloc)
        pltpu.sync_copy(adjT_hbm.at[b], aloc)
        pltpu.sync_copy(k1_hbm, k1loc)
        pltpu.sync_copy(k2_hbm, k2loc)

        @pl.when(q == 0)
        def _():
            pltpu.sync_copy(wd1_hbm, wd1loc)
            pltpu.sync_copy(bd1_hbm, bd1loc)
            pltpu.sync_copy(wd2_hbm, wd2loc)
            pltpu.sync_copy(bd2_hbm, bd2loc)

        def s1(i, c):
            l0 = z16
            l1 = z16
            for lane in range(16):
                f = i * 16 + lane
                l0 = l0 + xloc[f, pl.ds(nbase, 16)]
                l1 = l1 + xloc[f, pl.ds(nbase + 16, 16)]
            return (c[0] + l0, c[1] + l1)
        a0, a1 = lax.fori_loop(0, 8, s1, (z16, z16))
        xsbuf[0:16] = a0
        xsbuf[16:32] = a1
        pltpu.sync_copy(xsbuf, xs_sh.at[lb, pl.ds(nbase, NSLICE)])
        plsc.subcore_barrier()
        pltpu.sync_copy(xs_sh.at[lb], xs_b)

        def s2(i, c):
            v10, v11, v20, v21, sa = c
            xv = xs_b[pl.ds(i * 16, 16)]
            sa = sa + xv
            l10 = z16
            l11 = z16
            l20 = z16
            l21 = z16
            for lane in range(16):
                m = i * 16 + lane
                xm = _splat(xv, lane)
                r0 = aloc[m, pl.ds(nbase, 16)]
                r1 = aloc[m, pl.ds(nbase + 16, 16)]
                t0_ = r0 * xm
                t1_ = r1 * xm
                l10 = l10 + t0_
                l11 = l11 + t1_
                l20 = l20 + r0 * t0_
                l21 = l21 + r1 * t1_
            return (v10 + l10, v11 + l11, v20 + l20, v21 + l21, sa)
        v10, v11, v20, v21, sa = lax.fori_loop(0, 8, s2, (z16,) * 5)
        s0 = _hsum(sa)

        def s3(i, c):
            h0, h1 = c
            kav = k1loc[0, pl.ds(i * 16, 16)]
            kbv = k1loc[1, pl.ds(i * 16, 16)]
            kgv = k1loc[2, pl.ds(i * 16, 16)]
            lh0 = z16
            lh1 = z16
            for lane in range(16):
                ka = _splat(kav, lane)
                kb = _splat(kbv, lane)
                base = _splat(kgv, lane) * s0
                lh0 = lh0 + _tanh(v20 * ka + v10 * kb + base)
                lh1 = lh1 + _tanh(v21 * ka + v11 * kb + base)
            return (h0 + lh0, h1 + lh1)
        h0, h1 = lax.fori_loop(0, C1 // 16, s3, (z16, z16))
        h1buf[0:16] = h0
        h1buf[16:32] = h1
        pltpu.sync_copy(h1buf, h1s_sh.at[lb, pl.ds(nbase, NSLICE)])
        plsc.subcore_barrier()
        pltpu.sync_copy(h1s_sh.at[lb], h1s_b)

        def s4(i, c):
            w0, w1, ta = c
            hv = h1s_b[pl.ds(i * 16, 16)]
            ta = ta + hv
            lw0 = z16
            lw1 = z16
            for lane in range(16):
                m = i * 16 + lane
                hm = _splat(hv, lane)
                lw0 = lw0 + aloc[m, pl.ds(nbase, 16)] * hm
                lw1 = lw1 + aloc[m, pl.ds(nbase + 16, 16)] * hm
            return (w0 + lw0, w1 + lw1, ta)
        w0, w1, ta = lax.fori_loop(0, 8, s4, (z16,) * 3)
        t0 = _hsum(ta)
        wbuf[0:16] = w0
        wbuf[16:32] = w1

        def s5(i, c):
            wv = wbuf[pl.ds(i * 16, 16)]
            acc = list(c)
            for lane in range(16):
                wn = _splat(wv, lane)
                for j in range(8):
                    k20 = k2loc[0, 16 * j:16 * (j + 1)]
                    k21 = k2loc[1, 16 * j:16 * (j + 1)]
                    acc[j] = acc[j] + _tanh(k20 * wn + k21 * t0)
            return tuple(acc)
        pacc = lax.fori_loop(0, 2, s5, (z16,) * 8)
        for j in range(8):
            pbuf[16 * j:16 * (j + 1)] = pacc[j]
        pltpu.sync_copy(pbuf, pool_sh.at[sid])
        plsc.subcore_barrier()

        @pl.when(q == 0)
        def _():
            pltpu.sync_copy(pool_sh.at[sid], psum)
            for r in range(1, 4):
                pltpu.sync_copy(pool_sh.at[sid + r], ptmp)
                for j in range(8):
                    psum[16 * j:16 * (j + 1)] = (psum[16 * j:16 * (j + 1)]
                                                 + ptmp[16 * j:16 * (j + 1)])

            def dd(i, c):
                d0 = z16
                d1_ = z16
                d2 = z16
                d3 = z16
                pv = psum[pl.ds(i * 16, 16)]
                for lane in range(16):
                    kk = i * 16 + lane
                    pk = _splat(pv, lane)
                    d0 = d0 + wd1loc[kk, 0:16] * pk
                    d1_ = d1_ + wd1loc[kk, 16:32] * pk
                    d2 = d2 + wd1loc[kk, 32:48] * pk
                    d3 = d3 + wd1loc[kk, 48:64] * pk
                return (c[0] + d0, c[1] + d1_, c[2] + d2, c[3] + d3)
            d = lax.fori_loop(0, 8, dd, (z16,) * 4)
            for j in range(4):
                d1buf[16 * j:16 * (j + 1)] = _tanh(
                    d[j] + bd1loc[16 * j:16 * (j + 1)])

            def ll(i, acc):
                dv = d1buf[pl.ds(i * 16, 16)]
                for lane in range(16):
                    kk = i * 16 + lane
                    acc = acc + wd2loc[kk, 0:16] * _splat(dv, lane)
                return acc
            logits = lax.fori_loop(0, 4, ll, z16) + bd2loc[0:16]

            lane_i = lax.iota(jnp.int32, 16)
            mask = lane_i < 10
            ml = jnp.where(mask, logits, -1e30)
            mx = _hmax(ml)
            e = jnp.where(mask, jnp.exp(logits - mx), 0.0)
            s = _hsum(e)
            obuf[0:16] = e / s
            pltpu.sync_copy(obuf, out_hbm.at[b])

    return k(xT, adjT, k1t, k2t, Wd1, bd1, Wd2p, bd2p)


def kernel(x_batch, adj, k1, k2, Wd1, bd1, Wd2, bd2):
    xT = jnp.transpose(x_batch, (0, 2, 1))
    adjT = jnp.transpose(adj, (0, 2, 1))
    k1t = k1.T
    k2t = k2.T
    Wd2p = jnp.pad(Wd2, ((0, 0), (0, 6)))
    bd2p = jnp.pad(bd2, ((0, 6),))
    out = _sc_call(xT, adjT, k1t, k2t, Wd1, bd1, Wd2p, bd2p)
    return out[:, :10]
